# speedup vs baseline: 2.1845x; 2.1845x over previous
import functools

import jax
import jax.numpy as jnp
from jax import lax
from jax.experimental import pallas as pl
from jax.experimental.pallas import tpu as pltpu
from jax.experimental.pallas import tpu_sc as plsc

EPS = 1e-12

_NC = 2
_NS = 16
_NW = _NC * _NS
_CHUNK = 64


def _sc_gather(ids_flat, table):
    T = ids_flat.shape[0]
    H = table.shape[1]
    per_w = T // _NW
    n_chunks = per_w // _CHUNK
    mesh = plsc.VectorSubcoreMesh(core_axis_name="c", subcore_axis_name="s")

    @functools.partial(
        pl.kernel,
        out_type=jax.ShapeDtypeStruct((T, H), jnp.float32),
        mesh=mesh,
        scratch_types=[
            pltpu.VMEM((2, _CHUNK), jnp.int32),
            pltpu.VMEM((2, _CHUNK, H), jnp.float32),
            pltpu.SemaphoreType.DMA,
            pltpu.SemaphoreType.DMA,
            pltpu.SemaphoreType.DMA,
            pltpu.SemaphoreType.DMA,
        ],
    )
    def k(ids_hbm, tab_hbm, out_hbm, idx_v, rows_v, g0, g1, s0, s1):
        wid = lax.axis_index("s") * _NC + lax.axis_index("c")
        base = wid * per_w
        gsems = (g0, g1)
        ssems = (s0, s1)
        gcp = [None, None]
        scp = [None, None]
        for i in range(n_chunks):
            b = i % 2
            if scp[b] is not None:
                scp[b].wait()
            pltpu.sync_copy(ids_hbm.at[pl.ds(base + i * _CHUNK, _CHUNK)], idx_v.at[b])
            gcp[b] = pltpu.async_copy(tab_hbm.at[idx_v.at[b]], rows_v.at[b], gsems[b])
            if i > 0:
                pb = 1 - b
                gcp[pb].wait()
                scp[pb] = pltpu.async_copy(
                    rows_v.at[pb],
                    out_hbm.at[pl.ds(base + (i - 1) * _CHUNK, _CHUNK)],
                    ssems[pb],
                )
        last = n_chunks - 1
        lb = last % 2
        gcp[lb].wait()
        pltpu.sync_copy(rows_v.at[lb], out_hbm.at[pl.ds(base + last * _CHUNK, _CHUNK)])
        if scp[1 - lb] is not None:
            scp[1 - lb].wait()

    return k(ids_flat, table)


def _tc_ln(gath, tt2, ent2, pos_emb, type_emb, entity_emb, gamma2, beta2):
    T, H = gath.shape
    S = pos_emb.shape[0]
    BT = 256
    n_s = S // BT
    n_b = T // S

    def body(g_ref, tt_ref, ent_ref, pos_ref, te_ref, ee_ref, ga_ref, be_ref, o_ref):
        x = g_ref[...] + pos_ref[...]
        tt = tt_ref[...]
        ent = ent_ref[...]
        x = x + jnp.where(tt == 0, te_ref[0, :][None, :], te_ref[1, :][None, :])
        ec = jnp.where(
            ent == 0,
            ee_ref[0, :][None, :],
            jnp.where(
                ent == 1,
                ee_ref[1, :][None, :],
                jnp.where(ent == 2, ee_ref[2, :][None, :], ee_ref[3, :][None, :]),
            ),
        )
        x = x + ec
        mean = jnp.mean(x, axis=-1, keepdims=True)
        d = x - mean
        var = jnp.mean(d * d, axis=-1, keepdims=True)
        o_ref[...] = d * lax.rsqrt(var + EPS) * ga_ref[...] + be_ref[...]

    return pl.pallas_call(
        body,
        grid=(n_s, n_b),
        in_specs=[
            pl.BlockSpec((BT, H), lambda s, b: (b * n_s + s, 0)),
            pl.BlockSpec((BT, 1), lambda s, b: (b * n_s + s, 0)),
            pl.BlockSpec((BT, 1), lambda s, b: (b * n_s + s, 0)),
            pl.BlockSpec((BT, H), lambda s, b: (s, 0)),
            pl.BlockSpec((2, H), lambda s, b: (0, 0)),
            pl.BlockSpec((4, H), lambda s, b: (0, 0)),
            pl.BlockSpec((1, H), lambda s, b: (0, 0)),
            pl.BlockSpec((1, H), lambda s, b: (0, 0)),
        ],
        out_specs=pl.BlockSpec((BT, H), lambda s, b: (b * n_s + s, 0)),
        out_shape=jax.ShapeDtypeStruct((T, H), jnp.float32),
    )(gath, tt2, ent2, pos_emb, type_emb, entity_emb, gamma2, beta2)


def kernel(input_ids, entity_ids, token_type_ids, word_emb, pos_emb, type_emb, entity_emb, gamma, beta):
    B, S = input_ids.shape
    H = word_emb.shape[1]
    T = B * S
    ids = input_ids.reshape(T).astype(jnp.int32)
    gath = _sc_gather(ids, word_emb)
    tt2 = token_type_ids.reshape(T, 1).astype(jnp.int32)
    ent2 = entity_ids.reshape(T, 1).astype(jnp.int32)
    out = _tc_ln(
        gath, tt2, ent2, pos_emb, type_emb, entity_emb,
        gamma.reshape(1, H), beta.reshape(1, H),
    )
    return out.reshape(B, S, H)


# speedup vs baseline: 2.6179x; 1.1984x over previous
import functools

import jax
import jax.numpy as jnp
from jax import lax
from jax.experimental import pallas as pl
from jax.experimental.pallas import tpu as pltpu
from jax.experimental.pallas import tpu_sc as plsc

EPS = 1e-12

_NC = 2
_NS = 16
_NW = _NC * _NS
_CHUNK = 64


def _sc_gather(ids_flat, table):
    T = ids_flat.shape[0]
    H = table.shape[1]
    per_w = T // _NW
    n_chunks = per_w // _CHUNK
    mesh = plsc.VectorSubcoreMesh(core_axis_name="c", subcore_axis_name="s")

    @functools.partial(
        pl.kernel,
        out_type=jax.ShapeDtypeStruct((T, H), jnp.float32),
        mesh=mesh,
        scratch_types=[
            pltpu.VMEM((2, _CHUNK), jnp.int32),
            pltpu.VMEM((2, _CHUNK, H), jnp.float32),
            pltpu.SemaphoreType.DMA,
            pltpu.SemaphoreType.DMA,
            pltpu.SemaphoreType.DMA,
            pltpu.SemaphoreType.DMA,
        ],
    )
    def k(ids_hbm, tab_hbm, out_hbm, idx_v, rows_v, g0, g1, s0, s1):
        wid = lax.axis_index("s") * _NC + lax.axis_index("c")
        base = wid * per_w
        gsems = (g0, g1)
        ssems = (s0, s1)
        gcp = [None, None]
        scp = [None, None]
        for i in range(n_chunks):
            b = i % 2
            if scp[b] is not None:
                scp[b].wait()
            pltpu.sync_copy(ids_hbm.at[pl.ds(base + i * _CHUNK, _CHUNK)], idx_v.at[b])
            gcp[b] = pltpu.async_copy(tab_hbm.at[idx_v.at[b]], rows_v.at[b], gsems[b])
            if i > 0:
                pb = 1 - b
                gcp[pb].wait()
                scp[pb] = pltpu.async_copy(
                    rows_v.at[pb],
                    out_hbm.at[pl.ds(base + (i - 1) * _CHUNK, _CHUNK)],
                    ssems[pb],
                )
        last = n_chunks - 1
        lb = last % 2
        gcp[lb].wait()
        pltpu.sync_copy(rows_v.at[lb], out_hbm.at[pl.ds(base + last * _CHUNK, _CHUNK)])
        if scp[1 - lb] is not None:
            scp[1 - lb].wait()

    return k(ids_flat, table)


def _tc_ln(gath, tt2, ent2, pos_emb, type_emb, entity_emb, gamma2, beta2):
    T, H = gath.shape
    S = pos_emb.shape[0]
    BT = 512
    n_s = S // BT
    n_b = T // S

    def body(g_ref, tt_ref, ent_ref, pos_ref, te_ref, ee_ref, ga_ref, be_ref, o_ref):
        comb8 = jnp.concatenate(
            [te_ref[0, :][None, :] + ee_ref[...], te_ref[1, :][None, :] + ee_ref[...]],
            axis=0,
        )
        idx8 = tt_ref[...] * 4 + ent_ref[...]
        onehot = (idx8 == lax.broadcasted_iota(jnp.int32, (1, 8), 1)).astype(jnp.float32)
        x = g_ref[...] + pos_ref[...] + jnp.dot(
            onehot, comb8, preferred_element_type=jnp.float32
        )
        mean = jnp.mean(x, axis=-1, keepdims=True)
        d = x - mean
        var = jnp.mean(d * d, axis=-1, keepdims=True)
        o_ref[...] = d * lax.rsqrt(var + EPS) * ga_ref[...] + be_ref[...]

    return pl.pallas_call(
        body,
        grid=(n_s, n_b),
        in_specs=[
            pl.BlockSpec((BT, H), lambda s, b: (b * n_s + s, 0)),
            pl.BlockSpec((BT, 1), lambda s, b: (b * n_s + s, 0)),
            pl.BlockSpec((BT, 1), lambda s, b: (b * n_s + s, 0)),
            pl.BlockSpec((BT, H), lambda s, b: (s, 0)),
            pl.BlockSpec((2, H), lambda s, b: (0, 0)),
            pl.BlockSpec((4, H), lambda s, b: (0, 0)),
            pl.BlockSpec((1, H), lambda s, b: (0, 0)),
            pl.BlockSpec((1, H), lambda s, b: (0, 0)),
        ],
        out_specs=pl.BlockSpec((BT, H), lambda s, b: (b * n_s + s, 0)),
        out_shape=jax.ShapeDtypeStruct((T, H), jnp.float32),
    )(gath, tt2, ent2, pos_emb, type_emb, entity_emb, gamma2, beta2)


def kernel(input_ids, entity_ids, token_type_ids, word_emb, pos_emb, type_emb, entity_emb, gamma, beta):
    B, S = input_ids.shape
    H = word_emb.shape[1]
    T = B * S
    ids = input_ids.reshape(T).astype(jnp.int32)
    gath = _sc_gather(ids, word_emb)
    tt2 = token_type_ids.reshape(T, 1).astype(jnp.int32)
    ent2 = entity_ids.reshape(T, 1).astype(jnp.int32)
    out = _tc_ln(
        gath, tt2, ent2, pos_emb, type_emb, entity_emb,
        gamma.reshape(1, H), beta.reshape(1, H),
    )
    return out.reshape(B, S, H)
